# baseline (speedup 1.0000x reference)
import functools

import jax
import jax.numpy as jnp
from jax import lax
from jax.experimental import pallas as pl
from jax.experimental.pallas import tpu as pltpu
from jax.experimental.pallas import tpu_sc as plsc

VOCAB = 1000000
EMB = 64
BATCH = 4096
SEQ = 200

NW = 32
BPW = BATCH // NW
NPAIR = SEQ // 2
LANES = 16

_mesh = plsc.VectorSubcoreMesh(core_axis_name="c", subcore_axis_name="s")


def _transpose_tile(rbuf, tbuf):
    iota = lax.iota(jnp.int32, LANES)

    def erow(e, carry):
        col = jnp.full((LANES,), e, dtype=jnp.int32)
        for k in range(BPW // LANES):
            rows = iota + (k * LANES)
            vals = plsc.load_gather(rbuf, [rows, col])
            tbuf[e, pl.ds(k * LANES, LANES)] = vals
        return carry

    lax.fori_loop(0, EMB, erow, 0)


@functools.partial(
    pl.kernel,
    out_type=jax.ShapeDtypeStruct((SEQ, EMB, BATCH), jnp.float32),
    mesh=_mesh,
    scratch_types=[
        pltpu.VMEM((SEQ, BPW), jnp.int32),
        pltpu.VMEM((BPW, EMB), jnp.float32),
        pltpu.VMEM((BPW, EMB), jnp.float32),
        pltpu.VMEM((EMB, BPW), jnp.float32),
        pltpu.VMEM((EMB, BPW), jnp.float32),
        pltpu.SemaphoreType.DMA,
        pltpu.SemaphoreType.DMA,
        pltpu.SemaphoreType.DMA,
        pltpu.SemaphoreType.DMA,
    ],
    compiler_params=pltpu.CompilerParams(
        use_tc_tiling_on_sc=False,
        vmem_limit_bytes=4 * 1024 * 1024,
        needs_layout_passes=False,
    ),
)
def _embed_gather(idx_hbm, table_hbm, out_hbm, idx_v, rbuf0, rbuf1,
                  tbuf0, tbuf1, gsem0, gsem1, ssem0, ssem1):
    cid = lax.axis_index("c")
    sid = lax.axis_index("s")
    wid = sid * 2 + cid
    bbase = wid * BPW

    pltpu.sync_copy(idx_hbm.at[wid], idx_v)

    def gather(s, rbuf, gsem):
        return pltpu.make_async_copy(table_hbm.at[idx_v.at[s]], rbuf, gsem)

    def store(s, tbuf, ssem):
        return pltpu.make_async_copy(
            tbuf, out_hbm.at[s, :, pl.ds(bbase, BPW)], ssem
        )

    gather(0, rbuf0, gsem0).start()

    def body(t, carry):
        s0 = 2 * t
        s1 = s0 + 1

        gather(s1, rbuf1, gsem1).start()

        @pl.when(t > 0)
        def _():
            store(s0, tbuf0, ssem0).wait()
        gather(s0, rbuf0, gsem0).wait()
        _transpose_tile(rbuf0, tbuf0)
        store(s0, tbuf0, ssem0).start()

        @pl.when(t < NPAIR - 1)
        def _():
            gather(s0 + 2, rbuf0, gsem0).start()

        @pl.when(t > 0)
        def _():
            store(s1, tbuf1, ssem1).wait()
        gather(s1, rbuf1, gsem1).wait()
        _transpose_tile(rbuf1, tbuf1)
        store(s1, tbuf1, ssem1).start()
        return carry

    lax.fori_loop(0, NPAIR, body, 0)

    store(SEQ - 2, tbuf0, ssem0).wait()
    store(SEQ - 1, tbuf1, ssem1).wait()


def kernel(inp, hidden, embed_weight):
    del hidden
    idx = inp.astype(jnp.int32).T.reshape(SEQ, NW, BPW).transpose(1, 0, 2)
    out_t = _embed_gather(idx, embed_weight)
    return out_t.transpose(2, 0, 1)


# speedup vs baseline: 1.0020x; 1.0020x over previous
import functools

import jax
import jax.numpy as jnp
from jax import lax
from jax.experimental import pallas as pl
from jax.experimental.pallas import tpu as pltpu
from jax.experimental.pallas import tpu_sc as plsc

VOCAB = 1000000
EMB = 64
BATCH = 4096
SEQ = 200

NW = 32
BPW = BATCH // NW
NPAIR = SEQ // 2
LANES = 16
ETILES = EMB // 8

_mesh = plsc.VectorSubcoreMesh(core_axis_name="c", subcore_axis_name="s")


def _transpose_tile(rbuf, tbuf):
    iota = lax.iota(jnp.int32, LANES)
    rows = [iota + k * LANES for k in range(BPW // LANES)]
    for e in range(EMB):
        col = jnp.full((LANES,), e, dtype=jnp.int32)
        for k in range(BPW // LANES):
            tbuf[e, pl.ds(k * LANES, LANES)] = plsc.load_gather(
                rbuf, [rows[k], col]
            )


def _transpose_idx(src, dst):
    iota = lax.iota(jnp.int32, LANES)

    def body(s, carry):
        col = jnp.full((LANES,), s, dtype=jnp.int32)
        for k in range(BPW // LANES):
            dst[s, pl.ds(k * LANES, LANES)] = plsc.load_gather(
                src, [iota + k * LANES, col]
            )
        return carry

    lax.fori_loop(0, SEQ, body, 0)


@functools.partial(
    pl.kernel,
    out_type=jax.ShapeDtypeStruct((SEQ * ETILES, NW, 8, BPW), jnp.float32),
    mesh=_mesh,
    scratch_types=[
        pltpu.VMEM((BPW, SEQ), jnp.int32),
        pltpu.VMEM((SEQ, BPW), jnp.int32),
        pltpu.VMEM((BPW, EMB), jnp.float32),
        pltpu.VMEM((BPW, EMB), jnp.float32),
        pltpu.VMEM((EMB, BPW), jnp.float32),
        pltpu.VMEM((EMB, BPW), jnp.float32),
        pltpu.SemaphoreType.DMA,
        pltpu.SemaphoreType.DMA,
        pltpu.SemaphoreType.DMA,
        pltpu.SemaphoreType.DMA,
    ],
    compiler_params=pltpu.CompilerParams(
        use_tc_tiling_on_sc=False,
        vmem_limit_bytes=4 * 1024 * 1024,
        needs_layout_passes=False,
    ),
)
def _embed_gather(idx_hbm, table_hbm, out_hbm, idx_stage, idx_v,
                  rbuf0, rbuf1, tbuf0, tbuf1, gsem0, gsem1, ssem0, ssem1):
    cid = lax.axis_index("c")
    sid = lax.axis_index("s")
    wid = sid * 2 + cid

    pltpu.sync_copy(idx_hbm.at[pl.ds(wid * BPW, BPW)], idx_stage)
    _transpose_idx(idx_stage, idx_v)

    def gather(s, rbuf, gsem):
        return pltpu.make_async_copy(table_hbm.at[idx_v.at[s]], rbuf, gsem)

    def stores(s, tbuf, ssem):
        return [
            pltpu.make_async_copy(
                tbuf.at[pl.ds(et * 8, 8)],
                out_hbm.at[s * ETILES + et, wid],
                ssem,
            )
            for et in range(ETILES)
        ]

    gather(0, rbuf0, gsem0).start()

    def body(t, carry):
        s0 = 2 * t
        s1 = s0 + 1

        gather(s1, rbuf1, gsem1).start()

        @pl.when(t > 0)
        def _():
            for st in stores(s0, tbuf0, ssem0):
                st.wait()
        gather(s0, rbuf0, gsem0).wait()
        _transpose_tile(rbuf0, tbuf0)
        for st in stores(s0, tbuf0, ssem0):
            st.start()

        @pl.when(t < NPAIR - 1)
        def _():
            gather(s0 + 2, rbuf0, gsem0).start()

        @pl.when(t > 0)
        def _():
            for st in stores(s1, tbuf1, ssem1):
                st.wait()
        gather(s1, rbuf1, gsem1).wait()
        _transpose_tile(rbuf1, tbuf1)
        for st in stores(s1, tbuf1, ssem1):
            st.start()
        return carry

    lax.fori_loop(0, NPAIR, body, 0)

    for st in stores(SEQ - 2, tbuf0, ssem0):
        st.wait()
    for st in stores(SEQ - 1, tbuf1, ssem1):
        st.wait()


def kernel(inp, hidden, embed_weight):
    del hidden
    out_t = _embed_gather(inp.astype(jnp.int32), embed_weight)
    out5 = out_t.reshape(SEQ, ETILES, NW, 8, BPW)
    return out5.transpose(2, 4, 0, 1, 3).reshape(BATCH, SEQ, EMB)


# speedup vs baseline: 2.0411x; 2.0370x over previous
import functools

import jax
import jax.numpy as jnp
from jax import lax
from jax.experimental import pallas as pl
from jax.experimental.pallas import tpu as pltpu
from jax.experimental.pallas import tpu_sc as plsc

VOCAB = 1000000
EMB = 64
BATCH = 4096
SEQ = 200

NW = 32
BPW = BATCH // NW
NPAIR = SEQ // 2
LANES = 16
ETILES = EMB // 8
TPAD = BPW + 1

_mesh = plsc.VectorSubcoreMesh(core_axis_name="c", subcore_axis_name="s")


def _transpose_tile(rbuf, tbuf):
    iota = lax.iota(jnp.int32, LANES)
    erows = [iota + k * LANES for k in range(EMB // LANES)]

    def brow(b, carry):
        col = jnp.full((LANES,), b, dtype=jnp.int32)
        for k in range(EMB // LANES):
            vals = rbuf[b, pl.ds(k * LANES, LANES)]
            plsc.store_scatter(tbuf, [erows[k], col], vals)
        return carry

    lax.fori_loop(0, BPW, brow, 0, unroll=8)


@functools.partial(
    pl.kernel,
    out_type=jax.ShapeDtypeStruct((SEQ * ETILES, NW, 8, BPW), jnp.float32),
    mesh=_mesh,
    scratch_types=[
        pltpu.VMEM((BPW * SEQ,), jnp.int32),
        pltpu.VMEM((SEQ, BPW), jnp.int32),
        pltpu.VMEM((BPW, EMB), jnp.float32),
        pltpu.VMEM((BPW, EMB), jnp.float32),
        pltpu.VMEM((EMB, TPAD), jnp.float32),
        pltpu.VMEM((EMB, TPAD), jnp.float32),
        pltpu.SemaphoreType.DMA,
        pltpu.SemaphoreType.DMA,
        pltpu.SemaphoreType.DMA,
        pltpu.SemaphoreType.DMA,
    ],
    compiler_params=pltpu.CompilerParams(
        use_tc_tiling_on_sc=False,
        vmem_limit_bytes=4 * 1024 * 1024,
        needs_layout_passes=False,
    ),
)
def _embed_gather(idx_hbm, table_hbm, out_hbm, idx_stage, idx_v,
                  rbuf0, rbuf1, tbuf0, tbuf1, gsem0, gsem1, ssem0, ssem1):
    cid = lax.axis_index("c")
    sid = lax.axis_index("s")
    wid = sid * 2 + cid

    pltpu.sync_copy(idx_hbm.at[pl.ds(wid * BPW * SEQ, BPW * SEQ)], idx_stage)
    iota = lax.iota(jnp.int32, LANES)
    brows = [(iota + k * LANES) * SEQ for k in range(BPW // LANES)]

    def idx_t(s, carry):
        for k in range(BPW // LANES):
            idx_v[s, pl.ds(k * LANES, LANES)] = plsc.load_gather(
                idx_stage, [brows[k] + s]
            )
        return carry

    lax.fori_loop(0, SEQ, idx_t, 0)

    def gather(s, rbuf, gsem):
        return pltpu.make_async_copy(table_hbm.at[idx_v.at[s]], rbuf, gsem)

    def stores(s, tbuf, ssem):
        return [
            pltpu.make_async_copy(
                tbuf.at[pl.ds(et * 8, 8), pl.ds(0, BPW)],
                out_hbm.at[s * ETILES + et, wid],
                ssem,
            )
            for et in range(ETILES)
        ]

    gather(0, rbuf0, gsem0).start()

    def body(t, carry):
        s0 = 2 * t
        s1 = s0 + 1

        gather(s1, rbuf1, gsem1).start()

        @pl.when(t > 0)
        def _():
            for st in stores(s0, tbuf0, ssem0):
                st.wait()
        gather(s0, rbuf0, gsem0).wait()
        _transpose_tile(rbuf0, tbuf0)
        for st in stores(s0, tbuf0, ssem0):
            st.start()

        @pl.when(t < NPAIR - 1)
        def _():
            gather(s0 + 2, rbuf0, gsem0).start()

        @pl.when(t > 0)
        def _():
            for st in stores(s1, tbuf1, ssem1):
                st.wait()
        gather(s1, rbuf1, gsem1).wait()
        _transpose_tile(rbuf1, tbuf1)
        for st in stores(s1, tbuf1, ssem1):
            st.start()
        return carry

    lax.fori_loop(0, NPAIR, body, 0)

    for st in stores(SEQ - 2, tbuf0, ssem0):
        st.wait()
    for st in stores(SEQ - 1, tbuf1, ssem1):
        st.wait()


def kernel(inp, hidden, embed_weight):
    del hidden
    out_t = _embed_gather(inp.astype(jnp.int32).reshape(-1), embed_weight)
    out5 = out_t.reshape(SEQ, ETILES, NW, 8, BPW)
    return out5.transpose(2, 4, 0, 1, 3).reshape(BATCH, SEQ, EMB)
